# baseline (device time: 20425 ns/iter reference)
import jax
import jax.numpy as jnp
from jax import lax
from jax.experimental import pallas as pl
from jax.experimental.pallas import tpu as pltpu

N_DEV = 8
SQ = 256
D = 1024
DH = 128
HQ_LOCAL = 8
GROUP = 4
SCALE = 0.08838834764831843
AXIS_XOR = (1, 3, 4)
PIECES = ((0, 384), (384, 384), (768, 256))
N_ROUNDS = 3
N_RB = 2
RB = SQ // N_RB


def kernel(x, Wq, Wo, Wk, Wv):
    return _attn_allreduce(x, Wq, Wk, Wv, Wo)


def _attn_allreduce(x, wq, wk, wv, wo):
    def body(x_ref, wq_ref, wk_ref, wv_ref, wo_ref, out_ref,
             x_s, wq_s, wk_s, wv_s, wo_s, out_s, in_sems,
             send_bufs, recv_bufs, send_sems, recv_sems):
        my_i = lax.axis_index("i")

        kv_lo = my_i * 2 * DH
        cp_x = pltpu.make_async_copy(x_ref.at[0], x_s, in_sems.at[0])
        cp_q = pltpu.make_async_copy(wq_ref, wq_s, in_sems.at[1])
        cp_k = pltpu.make_async_copy(
            wk_ref.at[:, pl.ds(kv_lo, 2 * DH)], wk_s, in_sems.at[2])
        cp_v = pltpu.make_async_copy(
            wv_ref.at[:, pl.ds(kv_lo, 2 * DH)], wv_s, in_sems.at[3])
        cp_wo = pltpu.make_async_copy(wo_ref, wo_s, in_sems.at[4])
        cp_x.start()
        cp_q.start()
        cp_k.start()
        cp_v.start()
        cp_wo.start()

        barrier = pltpu.get_barrier_semaphore()
        for px in AXIS_XOR:
            pl.semaphore_signal(
                barrier, inc=1,
                device_id=(my_i ^ px,),
                device_id_type=pl.DeviceIdType.MESH,
            )
        pl.semaphore_wait(barrier, 3)

        cp_x.wait()
        cp_k.wait()
        cp_v.wait()
        xb = x_s[:].astype(jnp.bfloat16)
        k = jnp.dot(xb, wk_s[:].astype(jnp.bfloat16),
                    preferred_element_type=jnp.float32).astype(jnp.bfloat16)
        v = jnp.dot(xb, wv_s[:].astype(jnp.bfloat16),
                    preferred_element_type=jnp.float32).astype(jnp.bfloat16)
        cp_q.wait()
        wq_b = wq_s[:].astype(jnp.bfloat16)
        cp_wo.wait()

        def exchange(rb, p, r, piece):
            send_bufs[p][pl.ds(rb * RB, RB), :] = piece.astype(jnp.bfloat16)
            rdma = pltpu.make_async_remote_copy(
                src_ref=send_bufs[p].at[pl.ds(rb * RB, RB), :],
                dst_ref=recv_bufs[p].at[r, pl.ds(rb * RB, RB), :],
                send_sem=send_sems.at[rb, p, r],
                recv_sem=recv_sems.at[rb, p, r],
                device_id=(my_i ^ AXIS_XOR[(p + r) % 3],),
                device_id_type=pl.DeviceIdType.MESH,
            )
            rdma.start()
            return rdma

        pieces = [[None] * 3 for _ in range(N_RB)]
        rdmas = [[None] * 3 for _ in range(N_RB)]
        for rb in range(N_RB):
            rows = slice(rb * RB, (rb + 1) * RB)
            q = jnp.dot(xb[rows], wq_b,
                        preferred_element_type=jnp.float32)
            heads = []
            for h in range(HQ_LOCAL):
                qh = q[:, h * DH:(h + 1) * DH].astype(jnp.bfloat16)
                g = h // GROUP
                kg = k[:, g * DH:(g + 1) * DH]
                vg = v[:, g * DH:(g + 1) * DH]
                s = lax.dot_general(qh, kg, (((1,), (1,)), ((), ())),
                                    preferred_element_type=jnp.float32) * SCALE
                m = jnp.max(s, axis=1, keepdims=True)
                p_ = jnp.exp(s - m)
                l = jnp.sum(p_, axis=1, keepdims=True)
                o = jnp.dot(p_.astype(jnp.bfloat16), vg,
                            preferred_element_type=jnp.float32) / l
                heads.append(o)
            attn = jnp.concatenate(heads, axis=1).astype(jnp.bfloat16)
            for p, (o, w) in enumerate(PIECES):
                piece = jnp.dot(attn, wo_s[:, o:o + w].astype(jnp.bfloat16),
                                preferred_element_type=jnp.float32
                                ).astype(jnp.bfloat16)
                pieces[rb][p] = piece
                rdmas[rb][p] = exchange(rb, p, 0, piece)

        cp_outs = []
        for r in range(N_ROUNDS):
            for rb in range(N_RB):
                rows = pl.ds(rb * RB, RB)
                for p, (o, w) in enumerate(PIECES):
                    rdmas[rb][p].wait()
                    acc = pieces[rb][p] + recv_bufs[p][r, rows, :]
                    pieces[rb][p] = acc
                    if r + 1 < N_ROUNDS:
                        rdmas[rb][p] = exchange(rb, p, r + 1, acc)
                    else:
                        out_s[rows, o:o + w] = acc.astype(jnp.float32)
                if r + 1 == N_ROUNDS:
                    cp = pltpu.make_async_copy(
                        out_s.at[rows, :], out_ref.at[0, rows, :],
                        in_sems.at[5 + rb])
                    cp.start()
                    cp_outs.append(cp)
        for cp in cp_outs:
            cp.wait()

    return pl.pallas_call(
        body,
        out_shape=jax.ShapeDtypeStruct((1, SQ, D), jnp.float32),
        in_specs=[pl.BlockSpec(memory_space=pltpu.MemorySpace.HBM)] * 5,
        out_specs=pl.BlockSpec(memory_space=pltpu.MemorySpace.HBM),
        scratch_shapes=[
            pltpu.VMEM((SQ, D), jnp.float32),
            pltpu.VMEM((D, D), jnp.float32),
            pltpu.VMEM((D, 2 * DH), jnp.float32),
            pltpu.VMEM((D, 2 * DH), jnp.float32),
            pltpu.VMEM((D, D), jnp.float32),
            pltpu.VMEM((SQ, D), jnp.float32),
            pltpu.SemaphoreType.DMA((5 + N_RB,)),
            [pltpu.VMEM((SQ, w), jnp.bfloat16) for _, w in PIECES],
            [pltpu.VMEM((N_ROUNDS, SQ, w), jnp.bfloat16) for _, w in PIECES],
            pltpu.SemaphoreType.DMA((N_RB, 3, N_ROUNDS)),
            pltpu.SemaphoreType.DMA((N_RB, 3, N_ROUNDS)),
        ],
        compiler_params=pltpu.CompilerParams(collective_id=0),
    )(*[pltpu.with_memory_space_constraint(a, pltpu.MemorySpace.HBM)
        for a in (x, wq, wk, wv, wo)])
